# baseline (speedup 1.0000x reference)
import functools

import jax
import jax.numpy as jnp
from jax.experimental import pallas as pl
from jax.experimental.pallas import tpu as pltpu
from jax.experimental.pallas import tpu_sc as plsc

_B, _N = 4096, 128
_R = _B * _N
_KIN_DIM = 128
_EMB_DIM = 64
_BB = 32
_BR = _BB * _N

_NW = 32
_RW = _R // _NW
_CH = 256
_NCH = _RW // _CH
_NBUF = 3
_LOOK = 2

_sc_mesh = plsc.VectorSubcoreMesh(core_axis_name="c", subcore_axis_name="s")


@functools.partial(
    pl.kernel,
    out_type=jax.ShapeDtypeStruct((_R, 256), jnp.float32),
    mesh=_sc_mesh,
    scratch_types=[
        pltpu.VMEM((_RW,), jnp.int32),
        pltpu.VMEM((_NBUF, _CH, 128), jnp.float32),
        pltpu.SemaphoreType.DMA,
        pltpu.SemaphoreType.DMA,
    ],
)
def _sc_emb(key_hbm, tab_hbm, out_hbm, idv, rows, sem_g, sem_o):
    c = jax.lax.axis_index("c")
    s = jax.lax.axis_index("s")
    base = (s * 2 + c) * _RW

    pltpu.sync_copy(key_hbm.at[pl.ds(base, _RW)], idv)

    def gather(j, buf):
        idx = idv.at[pl.ds(j * _CH, _CH)]
        return pltpu.make_async_copy(tab_hbm.at[idx], rows.at[buf], sem_g)

    def outcp(j, buf):
        dst = out_hbm.at[pl.ds(base + j * _CH, _CH), pl.ds(128, 128)]
        return pltpu.make_async_copy(rows.at[buf], dst, sem_o)

    for j in range(_LOOK):
        gather(j, j % _NBUF).start()

    def step(j, carry):
        b = jax.lax.rem(j, _NBUF)
        bn = jax.lax.rem(j + _LOOK, _NBUF)
        gather(j, b).wait()
        outcp(j, b).start()

        @pl.when(j >= _NBUF - _LOOK)
        def _():
            outcp(j - (_NBUF - _LOOK), bn).wait()

        @pl.when(j + _LOOK < _NCH)
        def _():
            gather(j + _LOOK, bn).start()

        return carry

    jax.lax.fori_loop(0, _NCH, step, 0)
    for t in range(_NBUF - _LOOK):
        j = _NCH - (_NBUF - _LOOK) + t
        outcp(j, (j + _LOOK) % _NBUF).wait()


def _tc_body(kin_ref, w_ref, b_ref, emb_ref, out_ref):
    del emb_ref
    for i in range(_BB):
        kin_emb = jax.lax.dot_general(
            kin_ref[i], w_ref[...], (((0,), (0,)), ((), ())),
            preferred_element_type=jnp.float32)
        out_ref[i * _N:(i + 1) * _N, :] = kin_emb + b_ref[...]


@functools.partial(jax.jit, static_argnames=("interpret",))
def _run(kinematics, particle_ids, charges, W, b, pid_table, charge_table,
         interpret=False):
    kin_t = jnp.transpose(kinematics, (0, 2, 1))
    ids = particle_ids.reshape(_R)
    ch = charges.reshape(_R)
    b2 = b.reshape(1, _KIN_DIM)
    k = jnp.arange(60)
    ctab = jnp.zeros((64, 2 * _EMB_DIM), jnp.float32)
    ctab = ctab.at[:60, :_EMB_DIM].set(pid_table[k // 3])
    ctab = ctab.at[:60, _EMB_DIM:].set(charge_table[k % 3])
    keys = ids * 3 + ch + 1

    emb_out = _sc_emb(keys, ctab)

    out = pl.pallas_call(
        _tc_body,
        grid=(_B // _BB,),
        in_specs=[
            pl.BlockSpec((_BB, 4, _N), lambda i: (i, 0, 0)),
            pl.BlockSpec((4, _KIN_DIM), lambda i: (0, 0)),
            pl.BlockSpec((1, _KIN_DIM), lambda i: (0, 0)),
            pl.BlockSpec(memory_space=pltpu.MemorySpace.HBM),
        ],
        out_specs=pl.BlockSpec((_BR, _KIN_DIM), lambda i: (i, 0)),
        out_shape=jax.ShapeDtypeStruct((_R, 256), jnp.float32),
        input_output_aliases={3: 0},
        compiler_params=pltpu.CompilerParams(
            dimension_semantics=("parallel",)),
        interpret=interpret,
    )(kin_t, W, b2, emb_out)
    return out.reshape(_B, _N, 256)


def kernel(kinematics, particle_ids, charges, W, b, pid_table, charge_table):
    return _run(kinematics, particle_ids, charges, W, b, pid_table,
                charge_table)


# speedup vs baseline: 7.5922x; 7.5922x over previous
import functools

import jax
import jax.numpy as jnp
from jax.experimental import pallas as pl
from jax.experimental.pallas import tpu as pltpu

_B, _N = 4096, 128
_R = _B * _N
_KIN_DIM = 128
_EMB_DIM = 64
_VOC = 32
_BB = 64
_BR = _BB * _N


def _body(kin_ref, ids_ref, ch_ref, w_ref, b_ref, tab_ref, out_ref):
    ids = ids_ref[...]
    ch = ch_ref[...]
    for i in range(_BB):
        kin_emb = jax.lax.dot_general(
            kin_ref[i], w_ref[...], (((0,), (0,)), ((), ())),
            preferred_element_type=jnp.float32)
        out_ref[i * _N:(i + 1) * _N, 0:_KIN_DIM] = kin_emb + b_ref[...]
    rows = jax.lax.broadcasted_iota(jnp.int32, (_VOC, _BR), 0)
    ids_b = jnp.broadcast_to(ids[None, :], (_VOC, _BR))
    chp_b = jnp.broadcast_to((ch + 21)[None, :], (_VOC, _BR))
    oh = (rows == jnp.where(rows < 20, ids_b, chp_b)).astype(jnp.float32)
    emb = jax.lax.dot_general(
        oh, tab_ref[...], (((0,), (0,)), ((), ())),
        preferred_element_type=jnp.float32)
    out_ref[:, _KIN_DIM:] = emb


@functools.partial(jax.jit, static_argnames=("interpret",))
def _run(kinematics, particle_ids, charges, W, b, pid_table, charge_table,
         interpret=False):
    kin_t = jnp.transpose(kinematics, (0, 2, 1))
    ids = particle_ids.reshape(_R)
    ch = charges.reshape(_R)
    b2 = b.reshape(1, _KIN_DIM)
    tab = jnp.zeros((_VOC, 2 * _EMB_DIM), jnp.float32)
    tab = tab.at[:20, :_EMB_DIM].set(pid_table)
    tab = tab.at[20:23, _EMB_DIM:].set(charge_table)
    grid = (_B // _BB,)
    out = pl.pallas_call(
        _body,
        grid=grid,
        in_specs=[
            pl.BlockSpec((_BB, 4, _N), lambda i: (i, 0, 0)),
            pl.BlockSpec((_BR,), lambda i: (i,)),
            pl.BlockSpec((_BR,), lambda i: (i,)),
            pl.BlockSpec((4, _KIN_DIM), lambda i: (0, 0)),
            pl.BlockSpec((1, _KIN_DIM), lambda i: (0, 0)),
            pl.BlockSpec((_VOC, 2 * _EMB_DIM), lambda i: (0, 0)),
        ],
        out_specs=pl.BlockSpec((_BR, 256), lambda i: (i, 0)),
        out_shape=jax.ShapeDtypeStruct((_R, 256), jnp.float32),
        compiler_params=pltpu.CompilerParams(
            dimension_semantics=("parallel",)),
        interpret=interpret,
    )(kin_t, ids, ch, W, b2, tab)
    return out.reshape(_B, _N, 256)


def kernel(kinematics, particle_ids, charges, W, b, pid_table, charge_table):
    return _run(kinematics, particle_ids, charges, W, b, pid_table,
                charge_table)


# speedup vs baseline: 7.7891x; 1.0259x over previous
import functools

import jax
import jax.numpy as jnp
from jax.experimental import pallas as pl
from jax.experimental.pallas import tpu as pltpu

_B, _N = 4096, 128
_R = _B * _N
_KIN_DIM = 128
_EMB_DIM = 64
_VOC = 32
_BB = 128
_BR = _BB * _N


def _body(kin_ref, ids_ref, ch_ref, w_ref, b_ref, tab_ref, out_ref):
    ids = ids_ref[...]
    ch = ch_ref[...]
    for i in range(_BB):
        kin_emb = jax.lax.dot_general(
            kin_ref[i], w_ref[...], (((0,), (0,)), ((), ())),
            preferred_element_type=jnp.float32)
        out_ref[i * _N:(i + 1) * _N, 0:_KIN_DIM] = kin_emb + b_ref[...]
    rows = jax.lax.broadcasted_iota(jnp.int32, (_VOC, _BR), 0)
    ids_b = jnp.broadcast_to(ids[None, :], (_VOC, _BR))
    chp_b = jnp.broadcast_to((ch + 21)[None, :], (_VOC, _BR))
    oh = (rows == jnp.where(rows < 20, ids_b, chp_b)).astype(jnp.float32)
    emb = jax.lax.dot_general(
        oh, tab_ref[...], (((0,), (0,)), ((), ())),
        preferred_element_type=jnp.float32)
    out_ref[:, _KIN_DIM:] = emb


@functools.partial(jax.jit, static_argnames=("interpret",))
def _run(kinematics, particle_ids, charges, W, b, pid_table, charge_table,
         interpret=False):
    kin_t = jnp.transpose(kinematics, (0, 2, 1))
    ids = particle_ids.reshape(_R)
    ch = charges.reshape(_R)
    b2 = b.reshape(1, _KIN_DIM)
    tab = jnp.zeros((_VOC, 2 * _EMB_DIM), jnp.float32)
    tab = tab.at[:20, :_EMB_DIM].set(pid_table)
    tab = tab.at[20:23, _EMB_DIM:].set(charge_table)
    grid = (_B // _BB,)
    out = pl.pallas_call(
        _body,
        grid=grid,
        in_specs=[
            pl.BlockSpec((_BB, 4, _N), lambda i: (i, 0, 0)),
            pl.BlockSpec((_BR,), lambda i: (i,)),
            pl.BlockSpec((_BR,), lambda i: (i,)),
            pl.BlockSpec((4, _KIN_DIM), lambda i: (0, 0)),
            pl.BlockSpec((1, _KIN_DIM), lambda i: (0, 0)),
            pl.BlockSpec((_VOC, 2 * _EMB_DIM), lambda i: (0, 0)),
        ],
        out_specs=pl.BlockSpec((_BR, 256), lambda i: (i, 0)),
        out_shape=jax.ShapeDtypeStruct((_R, 256), jnp.float32),
        compiler_params=pltpu.CompilerParams(
            dimension_semantics=("parallel",)),
        interpret=interpret,
    )(kin_t, ids, ch, W, b2, tab)
    return out.reshape(_B, _N, 256)


def kernel(kinematics, particle_ids, charges, W, b, pid_table, charge_table):
    return _run(kinematics, particle_ids, charges, W, b, pid_table,
                charge_table)
